# Initial kernel scaffold; baseline (speedup 1.0000x reference)
import jax
import jax.numpy as jnp
from jax.experimental import pallas as pl


def kernel(x, argmin, last, embed):
    raise NotImplementedError("write your pallas kernel here")


# speedup vs baseline: 2.2846x; 2.2846x over previous
import functools

import jax
import jax.numpy as jnp
from jax.experimental import pallas as pl


def _vq_body(x_ref, e_ref, res_ref, idx_ref, *, M, K, C):
    X = x_ref[0]
    E = e_ref[...]
    x2 = jnp.sum(X * X, axis=1, keepdims=True)
    e2 = jnp.sum(E * E, axis=1)[None, :]
    p = jax.lax.dot_general(X, E, (((1,), (1,)), ((), ())),
                            preferred_element_type=jnp.float32)
    d = (x2 - 2.0 * p) + e2

    colmin = jnp.min(d, axis=0, keepdims=True)
    gmin = jnp.min(colmin)
    kiota = jax.lax.broadcasted_iota(jnp.int32, (1, K), 1)
    k_star = jnp.min(jnp.where(colmin == gmin, kiota, K))
    dcol = jnp.min(jnp.where(kiota == k_star, d, jnp.inf),
                   axis=1, keepdims=True)
    miota = jax.lax.broadcasted_iota(jnp.int32, (M, 1), 0)
    m_star = jnp.min(jnp.where(dcol == gmin, miota, M))

    res_ref[0] = x_ref[0, pl.ds(m_star, 1), :] - e_ref[pl.ds(k_star, 1), :]
    idx_ref[0] = jnp.reshape(k_star, (1, 1))


def kernel(x, argmin, last, embed):
    del argmin
    N, M, C = x.shape
    K = embed.shape[0]
    body = functools.partial(_vq_body, M=M, K=K, C=C)
    res, idx = pl.pallas_call(
        body,
        grid=(N,),
        in_specs=[
            pl.BlockSpec((1, M, C), lambda n: (n, 0, 0)),
            pl.BlockSpec((K, C), lambda n: (0, 0)),
        ],
        out_specs=[
            pl.BlockSpec((1, 1, C), lambda n: (n, 0, 0)),
            pl.BlockSpec((1, 1, 1), lambda n: (n, 0, 0)),
        ],
        out_shape=[
            jax.ShapeDtypeStruct((N, 1, C), x.dtype),
            jax.ShapeDtypeStruct((N, 1, 1), jnp.int32),
        ],
    )(x, embed)
    return res * jnp.asarray(last, x.dtype), idx.reshape(N, 1)
